# baseline (speedup 1.0000x reference)
import jax
import jax.numpy as jnp
from jax import lax
from jax.experimental import pallas as pl
from jax.experimental.pallas import tpu as pltpu
from jax.experimental.pallas import tpu_sc as plsc

NU = 50000
NI = 50000
NN = NU + NI
EH = 16
NE = 1600000

NTILES = 16
CHUNK = 512
CROWS = CHUNK // 128
NCH = 196
HALF = NCH // 2
E_PT = NCH * CHUNK
PAD_E = NTILES * E_PT
S_ROWS = PAD_E // 128
T_ROWS = E_PT // 128

BLK = 400
NBLK = NN // BLK
NUB = NU // BLK
KMAX = 16

GROUPS = CHUNK // EH


def _sc_body(ue, ie, pk,
             x0f, lay1, lay2, uo, io,
             pk0, pk1, rows0, rows1, acc,
             semg0, semg1, semi):
    c = lax.axis_index("c")
    s = lax.axis_index("s")

    zero16 = jnp.zeros((EH,), jnp.float32)
    pkb = (pk0, pk1)
    rowsb = (rows0, rows1)
    semg = (semg0, semg1)
    cols = pl.ds(c * EH, EH)

    def blk_of(buf):
        return buf.at[pl.ds(0, BLK), :]

    def x0_dst(bid):
        return x0f.at[pl.ds(c * NN + bid * BLK, BLK), :]

    for k in range(KMAX):
        bid = s + k * NTILES

        @pl.when(bid < NBLK)
        def _():
            buf = blk_of(rowsb[k % 2])
            if k >= 2:
                pltpu.make_async_copy(buf, x0_dst(s + (k - 2) * NTILES),
                                      semi).wait()

            @pl.when(bid < NUB)
            def _():
                pltpu.sync_copy(ue.at[pl.ds(bid * BLK, BLK), cols], buf)

            @pl.when(bid >= NUB)
            def _():
                pltpu.sync_copy(ie.at[pl.ds(bid * BLK - NU, BLK), cols], buf)

            pltpu.async_copy(buf, x0_dst(bid), semi)

    for k in range(KMAX):
        bid = s + k * NTILES

        @pl.when((bid < NBLK) & (bid + 2 * NTILES >= NBLK))
        def _():
            pltpu.make_async_copy(blk_of(rowsb[k % 2]), x0_dst(bid),
                                  semi).wait()

    plsc.subcore_barrier()

    def idx_rec(i):
        return pk.at[s * NCH + i]

    def load_idx_sync(i, p):
        pltpu.sync_copy(idx_rec(i), pkb[p])

    def prefetch_idx(i, p):
        pltpu.async_copy(idx_rec(i), pkb[p], semi)

    def wait_idx(i, p):
        pltpu.make_async_copy(idx_rec(i), pkb[p], semi).wait()

    def wait_gather(p, j):
        pltpu.make_async_copy(
            x0f.at[pl.ds(0, 128), :],
            rowsb[p].at[pl.ds(j * 128, 128), :],
            semg[p],
        ).wait()

    def block_dmas(fn):
        @pl.loop(0, KMAX)
        def _(k):
            bid = s + k * NTILES

            @pl.when(bid < NBLK)
            def _():
                src_ref, dst_ref = fn(bid * BLK)
                pltpu.async_copy(src_ref, dst_ref, semi)

        @pl.loop(0, KMAX)
        def _(k):
            bid = s + k * NTILES

            @pl.when(bid < NBLK)
            def _():
                src_ref, dst_ref = fn(bid * BLK)
                pltpu.make_async_copy(src_ref, dst_ref, semi).wait()

    def do_layer(src_ref):
        view = src_ref.at[pl.ds(c * NN, NN)]

        @pl.loop(0, BLK)
        def _(i):
            rows0[i, :] = zero16

        block_dmas(lambda off: (
            blk_of(rows0), acc.at[pl.ds(off, BLK), :]))
        plsc.subcore_barrier()

        load_idx_sync(0, 0)
        for j in range(CROWS):
            pltpu.async_copy(
                view.at[pkb[0].at[j]],
                rowsb[0].at[pl.ds(j * 128, 128), :],
                semg[0],
            )
        prefetch_idx(1, 1)

        def half(kk, i, p, q):
            def with_next():
                wait_idx(i + 1, q)
                for j in range(CROWS):
                    wait_gather(p, j)
                    pltpu.async_copy(
                        view.at[pkb[q].at[j]],
                        rowsb[q].at[pl.ds(j * 128, 128), :],
                        semg[q],
                    )

            def without_next():
                for j in range(CROWS):
                    wait_gather(p, j)

            if p == 0:
                with_next()
            else:
                pl.when(kk < HALF - 1)(with_next)
                pl.when(kk >= HALF - 1)(without_next)

            rows_p = rowsb[p]
            pk_p = pkb[p]

            @pl.loop(0, GROUPS)
            def _(g):
                wg = plsc.bitcast(
                    pk_p[8 + g // 8, pl.ds((g % 8) * EH, EH)], jnp.float32)
                for jj in range(EH):
                    e = g * EH + jj
                    wj = lax.gather(
                        wg,
                        jnp.full((EH, 1), jj, jnp.int32),
                        lax.GatherDimensionNumbers(
                            offset_dims=(),
                            collapsed_slice_dims=(0,),
                            start_index_map=(0,),
                        ),
                        slice_sizes=(1,),
                        mode=lax.GatherScatterMode.PROMISE_IN_BOUNDS,
                    )
                    rows_p[e, :] = rows_p[e, :] * wj

            for j in range(CROWS):
                pltpu.sync_copy(
                    rows_p.at[pl.ds(j * 128, 128), :],
                    acc.at[pk_p.at[CROWS + j]],
                    add=True,
                )

            pl.when(kk < HALF - 1)(lambda: prefetch_idx(i + 2, p))

        @pl.loop(0, HALF)
        def _(kk):
            half(kk, 2 * kk, 0, 1)
            half(kk, 2 * kk + 1, 1, 0)

        plsc.subcore_barrier()

    def writeback(dst_hbm):
        block_dmas(lambda off: (
            acc.at[pl.ds(off, BLK), :],
            dst_hbm.at[pl.ds(c * NN + off, BLK), :],
        ))
        plsc.subcore_barrier()

    do_layer(x0f)
    writeback(lay1)
    do_layer(lay1)
    writeback(lay2)
    do_layer(lay2)

    def accum():
        @pl.loop(0, BLK)
        def _(r):
            rows0[r, :] = rows0[r, :] + rows1[r, :]

    for k in range(KMAX):
        bid = s + k * NTILES

        @pl.when(bid < NBLK)
        def _():
            noff = bid * BLK
            off = c * NN + noff
            pltpu.sync_copy(x0f.at[pl.ds(off, BLK), :], blk_of(rows0))
            pltpu.sync_copy(lay1.at[pl.ds(off, BLK), :], blk_of(rows1))
            accum()
            pltpu.sync_copy(lay2.at[pl.ds(off, BLK), :], blk_of(rows1))
            accum()
            pltpu.sync_copy(acc.at[pl.ds(noff, BLK), :], blk_of(rows1))
            accum()

            @pl.loop(0, BLK)
            def _(r):
                rows0[r, :] = rows0[r, :] * 0.25

            @pl.when(bid < NUB)
            def _():
                pltpu.sync_copy(blk_of(rows0),
                                uo.at[pl.ds(noff, BLK), cols])

            @pl.when(bid >= NUB)
            def _():
                pltpu.sync_copy(blk_of(rows0),
                                io.at[pl.ds(noff - NU, BLK), cols])


@jax.jit
def kernel(user_emb, item_emb, edge_index, edge_weight):
    dst = edge_index[0]
    src = edge_index[1]
    pad = PAD_E - NE
    fill = (jnp.arange(pad, dtype=jnp.int32) * 97) % NN
    src2 = jnp.concatenate([src, fill]).reshape(-1, CROWS, 128)
    dst2 = jnp.concatenate([dst, fill]).reshape(-1, CROWS, 128)
    w2 = lax.bitcast_convert_type(
        jnp.concatenate([edge_weight, jnp.zeros((pad,), jnp.float32)]),
        jnp.int32).reshape(-1, CROWS, 128)
    pk = jnp.concatenate([src2, dst2, w2], axis=1)

    mesh = plsc.VectorSubcoreMesh(
        core_axis_name="c", subcore_axis_name="s",
        num_cores=2, num_subcores=NTILES,
    )
    flat = jax.ShapeDtypeStruct((2 * NN, EH), jnp.float32)
    emb = jax.ShapeDtypeStruct((NU, 32), jnp.float32)
    sc = pl.kernel(
        _sc_body,
        out_type=(flat, flat, flat, emb, emb),
        mesh=mesh,
        scratch_types=[
            pltpu.VMEM((3 * CROWS, 128), jnp.int32),
            pltpu.VMEM((3 * CROWS, 128), jnp.int32),
            pltpu.VMEM((CHUNK, EH), jnp.float32),
            pltpu.VMEM((CHUNK, EH), jnp.float32),
            pltpu.VMEM_SHARED((NN, EH), jnp.float32),
            pltpu.SemaphoreType.DMA,
            pltpu.SemaphoreType.DMA,
            pltpu.SemaphoreType.DMA,
        ],
        compiler_params=pltpu.CompilerParams(
            use_tc_tiling_on_sc=False, needs_layout_passes=False),
    )
    _, _, _, uo, io = sc(user_emb, item_emb, pk)
    return uo, io


# speedup vs baseline: 1.0265x; 1.0265x over previous
import jax
import jax.numpy as jnp
from jax import lax
from jax.experimental import pallas as pl
from jax.experimental.pallas import tpu as pltpu
from jax.experimental.pallas import tpu_sc as plsc

NU = 50000
NI = 50000
NN = NU + NI
EH = 16
NE = 1600000

NTILES = 16
CHUNK = 512
CROWS = CHUNK // 128
NCH = 196
HALF = NCH // 2
E_PT = NCH * CHUNK
PAD_E = NTILES * E_PT
S_ROWS = PAD_E // 128
T_ROWS = E_PT // 128

BLK = 400
NBLK = NN // BLK
NUB = NU // BLK
KMAX = 16

GROUPS = CHUNK // EH


def _sc_body(ue, ie, src2, dst2, w2,
             x0f, lay1, lay2, uo, io,
             src0, src1, dst0, dst1, wv0, wv1, rows0, rows1, acc,
             semg0, semg1, semi):
    c = lax.axis_index("c")
    s = lax.axis_index("s")

    zero16 = jnp.zeros((EH,), jnp.float32)
    srcb = (src0, src1)
    dstb = (dst0, dst1)
    wb = (wv0, wv1)
    rowsb = (rows0, rows1)
    semg = (semg0, semg1)
    cols = pl.ds(c * EH, EH)

    def blk_of(buf):
        return buf.at[pl.ds(0, BLK), :]

    def x0_dst(bid):
        return x0f.at[pl.ds(c * NN + bid * BLK, BLK), :]

    for k in range(KMAX):
        bid = s + k * NTILES

        @pl.when(bid < NBLK)
        def _():
            buf = blk_of(rowsb[k % 2])
            if k >= 2:
                pltpu.make_async_copy(buf, x0_dst(s + (k - 2) * NTILES),
                                      semi).wait()

            @pl.when(bid < NUB)
            def _():
                pltpu.sync_copy(ue.at[pl.ds(bid * BLK, BLK), cols], buf)

            @pl.when(bid >= NUB)
            def _():
                pltpu.sync_copy(ie.at[pl.ds(bid * BLK - NU, BLK), cols], buf)

            pltpu.async_copy(buf, x0_dst(bid), semi)

    for k in range(KMAX):
        bid = s + k * NTILES

        @pl.when((bid < NBLK) & (bid + 2 * NTILES >= NBLK))
        def _():
            pltpu.make_async_copy(blk_of(rowsb[k % 2]), x0_dst(bid),
                                  semi).wait()

    plsc.subcore_barrier()

    def idx_row(arr, i):
        return arr.at[pl.ds(s * T_ROWS + i * CROWS, CROWS)]

    def load_idx_sync(i, p):
        pltpu.sync_copy(idx_row(src2, i), srcb[p])
        pltpu.sync_copy(idx_row(dst2, i), dstb[p])
        pltpu.sync_copy(idx_row(w2, i), wb[p])

    def prefetch_idx(i, p):
        pltpu.async_copy(idx_row(src2, i), srcb[p], semi)
        pltpu.async_copy(idx_row(dst2, i), dstb[p], semi)
        pltpu.async_copy(idx_row(w2, i), wb[p], semi)

    def wait_idx(i, p):
        pltpu.make_async_copy(idx_row(src2, i), srcb[p], semi).wait()
        pltpu.make_async_copy(idx_row(dst2, i), dstb[p], semi).wait()
        pltpu.make_async_copy(idx_row(w2, i), wb[p], semi).wait()

    def wait_gather(p, j):
        pltpu.make_async_copy(
            x0f.at[pl.ds(0, 128), :],
            rowsb[p].at[pl.ds(j * 128, 128), :],
            semg[p],
        ).wait()

    def block_dmas(fn):
        @pl.loop(0, KMAX)
        def _(k):
            bid = s + k * NTILES

            @pl.when(bid < NBLK)
            def _():
                src_ref, dst_ref = fn(bid * BLK)
                pltpu.async_copy(src_ref, dst_ref, semi)

        @pl.loop(0, KMAX)
        def _(k):
            bid = s + k * NTILES

            @pl.when(bid < NBLK)
            def _():
                src_ref, dst_ref = fn(bid * BLK)
                pltpu.make_async_copy(src_ref, dst_ref, semi).wait()

    def do_layer(src_ref):
        view = src_ref.at[pl.ds(c * NN, NN)]

        @pl.loop(0, BLK)
        def _(i):
            rows0[i, :] = zero16

        block_dmas(lambda off: (
            blk_of(rows0), acc.at[pl.ds(off, BLK), :]))
        plsc.subcore_barrier()

        load_idx_sync(0, 0)
        for j in range(CROWS):
            pltpu.async_copy(
                view.at[srcb[0].at[j]],
                rowsb[0].at[pl.ds(j * 128, 128), :],
                semg[0],
            )
        prefetch_idx(1, 1)

        def half(kk, i, p, q):
            def with_next():
                wait_idx(i + 1, q)
                for j in range(CROWS):
                    wait_gather(p, j)
                    pltpu.async_copy(
                        view.at[srcb[q].at[j]],
                        rowsb[q].at[pl.ds(j * 128, 128), :],
                        semg[q],
                    )

            def without_next():
                for j in range(CROWS):
                    wait_gather(p, j)

            if p == 0:
                with_next()
            else:
                pl.when(kk < HALF - 1)(with_next)
                pl.when(kk >= HALF - 1)(without_next)

            rows_p = rowsb[p]
            w_p = wb[p]

            @pl.loop(0, GROUPS)
            def _(g):
                wg = w_p[g // 8, pl.ds((g % 8) * EH, EH)]
                for jj in range(EH):
                    e = g * EH + jj
                    wj = lax.gather(
                        wg,
                        jnp.full((EH, 1), jj, jnp.int32),
                        lax.GatherDimensionNumbers(
                            offset_dims=(),
                            collapsed_slice_dims=(0,),
                            start_index_map=(0,),
                        ),
                        slice_sizes=(1,),
                        mode=lax.GatherScatterMode.PROMISE_IN_BOUNDS,
                    )
                    rows_p[e, :] = rows_p[e, :] * wj

            for j in range(CROWS):
                pltpu.sync_copy(
                    rows_p.at[pl.ds(j * 128, 128), :],
                    acc.at[dstb[p].at[j]],
                    add=True,
                )

            pl.when(kk < HALF - 1)(lambda: prefetch_idx(i + 2, p))

        @pl.loop(0, HALF)
        def _(kk):
            half(kk, 2 * kk, 0, 1)
            half(kk, 2 * kk + 1, 1, 0)

        plsc.subcore_barrier()

    def writeback(dst_hbm):
        block_dmas(lambda off: (
            acc.at[pl.ds(off, BLK), :],
            dst_hbm.at[pl.ds(c * NN + off, BLK), :],
        ))
        plsc.subcore_barrier()

    do_layer(x0f)
    writeback(lay1)
    do_layer(lay1)
    writeback(lay2)
    do_layer(lay2)

    def accum():
        @pl.loop(0, BLK)
        def _(r):
            rows0[r, :] = rows0[r, :] + rows1[r, :]

    for k in range(KMAX):
        bid = s + k * NTILES

        @pl.when(bid < NBLK)
        def _():
            noff = bid * BLK
            off = c * NN + noff
            pltpu.sync_copy(x0f.at[pl.ds(off, BLK), :], blk_of(rows0))
            pltpu.sync_copy(lay1.at[pl.ds(off, BLK), :], blk_of(rows1))
            accum()
            pltpu.sync_copy(lay2.at[pl.ds(off, BLK), :], blk_of(rows1))
            accum()
            pltpu.sync_copy(acc.at[pl.ds(noff, BLK), :], blk_of(rows1))
            accum()

            @pl.loop(0, BLK)
            def _(r):
                rows0[r, :] = rows0[r, :] * 0.25

            @pl.when(bid < NUB)
            def _():
                pltpu.sync_copy(blk_of(rows0),
                                uo.at[pl.ds(noff, BLK), cols])

            @pl.when(bid >= NUB)
            def _():
                pltpu.sync_copy(blk_of(rows0),
                                io.at[pl.ds(noff - NU, BLK), cols])


@jax.jit
def kernel(user_emb, item_emb, edge_index, edge_weight):
    dst = edge_index[0]
    src = edge_index[1]
    pad = PAD_E - NE
    fill = (jnp.arange(pad, dtype=jnp.int32) * 97) % NN
    src2 = jnp.concatenate([src, fill]).reshape(S_ROWS, 128)
    dst2 = jnp.concatenate([dst, fill]).reshape(S_ROWS, 128)
    w2 = jnp.concatenate(
        [edge_weight, jnp.zeros((pad,), jnp.float32)]).reshape(S_ROWS, 128)

    mesh = plsc.VectorSubcoreMesh(
        core_axis_name="c", subcore_axis_name="s",
        num_cores=2, num_subcores=NTILES,
    )
    flat = jax.ShapeDtypeStruct((2 * NN, EH), jnp.float32)
    emb = jax.ShapeDtypeStruct((NU, 32), jnp.float32)
    sc = pl.kernel(
        _sc_body,
        out_type=(flat, flat, flat, emb, emb),
        mesh=mesh,
        scratch_types=[
            pltpu.VMEM((CROWS, 128), jnp.int32),
            pltpu.VMEM((CROWS, 128), jnp.int32),
            pltpu.VMEM((CROWS, 128), jnp.int32),
            pltpu.VMEM((CROWS, 128), jnp.int32),
            pltpu.VMEM((CROWS, 128), jnp.float32),
            pltpu.VMEM((CROWS, 128), jnp.float32),
            pltpu.VMEM((CHUNK, EH), jnp.float32),
            pltpu.VMEM((CHUNK, EH), jnp.float32),
            pltpu.VMEM_SHARED((NN, EH), jnp.float32),
            pltpu.SemaphoreType.DMA,
            pltpu.SemaphoreType.DMA,
            pltpu.SemaphoreType.DMA,
        ],
        compiler_params=pltpu.CompilerParams(
            use_tc_tiling_on_sc=False, needs_layout_passes=False),
    )
    _, _, _, uo, io = sc(user_emb, item_emb, src2, dst2, w2)
    return uo, io
